# baseline (speedup 1.0000x reference)
import functools

import jax
import jax.numpy as jnp
from jax import lax
from jax.experimental import pallas as pl
from jax.experimental.pallas import tpu as pltpu
from jax.experimental.pallas import tpu_sc as plsc

NUM_WORKERS = 32
CHUNK_ROWS = 4


@functools.partial(jax.jit, static_argnames=("B", "F", "D"))
def _lookup_mean(flat_table, gidx, *, B, F, D):
    G = CHUNK_ROWS * F
    rows_per_w = B // NUM_WORKERS
    n_chunks = rows_per_w // CHUNK_ROWS
    inv_f = jnp.float32(1.0 / F)

    mesh = plsc.VectorSubcoreMesh(core_axis_name="c", subcore_axis_name="s")

    @functools.partial(
        pl.kernel,
        mesh=mesh,
        compiler_params=pltpu.CompilerParams(use_tc_tiling_on_sc=False),
        out_type=jax.ShapeDtypeStruct((B, D), jnp.float32),
        scratch_types=[
            pltpu.VMEM((G,), jnp.int32),
            pltpu.VMEM((G, D), jnp.float32),
            pltpu.VMEM((rows_per_w, D), jnp.float32),
            pltpu.SemaphoreType.DMA,
        ],
    )
    def k(table_hbm, gidx_hbm, out_hbm, idx_v, rows_v, out_v, sem):
        wid = lax.axis_index("s") * 2 + lax.axis_index("c")
        ibase = wid * (rows_per_w * F)

        def body(j, carry):
            pltpu.sync_copy(gidx_hbm.at[pl.ds(ibase + j * G, G)], idx_v)
            pltpu.async_copy(table_hbm.at[idx_v], rows_v, sem).wait()
            for b in range(CHUNK_ROWS):
                acc0 = rows_v[b * F, 0:16]
                acc1 = rows_v[b * F, 16:32]
                for f in range(1, F):
                    acc0 = acc0 + rows_v[b * F + f, 0:16]
                    acc1 = acc1 + rows_v[b * F + f, 16:32]
                row = j * CHUNK_ROWS + b
                out_v[row, 0:16] = acc0 * inv_f
                out_v[row, 16:32] = acc1 * inv_f
            return carry

        lax.fori_loop(0, n_chunks, body, 0)
        pltpu.sync_copy(out_v, out_hbm.at[pl.ds(wid * rows_per_w, rows_per_w)])

    return k(flat_table, gidx)


def kernel(categorical_vars_tensor, tables):
    F, V, D = tables.shape
    B = categorical_vars_tensor.shape[0]
    idx = categorical_vars_tensor.astype(jnp.int32)
    gidx = (idx + (jnp.arange(F, dtype=jnp.int32) * V)[None, :]).reshape(-1)
    flat_table = tables.reshape(F * V, D)
    return _lookup_mean(flat_table, gidx, B=B, F=F, D=D)


# speedup vs baseline: 1.1149x; 1.1149x over previous
import functools

import jax
import jax.numpy as jnp
from jax import lax
from jax.experimental import pallas as pl
from jax.experimental.pallas import tpu as pltpu
from jax.experimental.pallas import tpu_sc as plsc

NUM_WORKERS = 32
CB = 4
K = 8


@functools.partial(jax.jit, static_argnames=("B", "F", "D"))
def _lookup_mean(flat_table, gidx2, *, B, F, D):
    G = CB * F
    rows_per_w = B // NUM_WORKERS
    n_gathers = rows_per_w // CB
    n_super = n_gathers // K
    half = n_super // 2
    rows_per_super = K * CB
    inv_f = jnp.float32(1.0 / F)

    mesh = plsc.VectorSubcoreMesh(core_axis_name="c", subcore_axis_name="s")

    @functools.partial(
        pl.kernel,
        mesh=mesh,
        compiler_params=pltpu.CompilerParams(use_tc_tiling_on_sc=False),
        out_type=jax.ShapeDtypeStruct((B, D), jnp.float32),
        scratch_types=[
            pltpu.VMEM((n_gathers, G), jnp.int32),
            pltpu.VMEM((2 * K, G, D), jnp.float32),
            pltpu.VMEM((rows_per_w, D), jnp.float32),
            pltpu.SemaphoreType.DMA,
            pltpu.SemaphoreType.DMA,
        ],
    )
    def k(table_hbm, gidx_hbm, out_hbm, idx_v, rows_v, out_v, sem0, sem1):
        wid = lax.axis_index("s") * 2 + lax.axis_index("c")
        pltpu.sync_copy(gidx_hbm.at[pl.ds(wid * n_gathers, n_gathers)], idx_v)

        def fire(group, s, sem):
            for g in range(K):
                pltpu.async_copy(
                    table_hbm.at[idx_v.at[s * K + g]],
                    rows_v.at[group * K + g],
                    sem,
                )

        def drain(group, sem):
            for g in range(K):
                pltpu.make_async_copy(
                    table_hbm.at[idx_v.at[0]],
                    rows_v.at[group * K + g],
                    sem,
                ).wait()

        def accum(group, s):
            def row_body(i, carry):
                slot = group * K + lax.shift_right_logical(i, 2)
                b = lax.bitwise_and(i, 3) * F
                acc0 = rows_v[slot, b, 0:16]
                acc1 = rows_v[slot, b, 16:32]
                for f in range(1, F):
                    acc0 = acc0 + rows_v[slot, b + f, 0:16]
                    acc1 = acc1 + rows_v[slot, b + f, 16:32]
                row = s * rows_per_super + i
                out_v[row, 0:16] = acc0 * inv_f
                out_v[row, 16:32] = acc1 * inv_f
                return carry

            lax.fori_loop(0, rows_per_super, row_body, 0)

        fire(0, 0, sem0)
        fire(1, 1, sem1)

        def body(t, carry):
            s = 2 * t
            drain(0, sem0)
            accum(0, s)

            @pl.when(t < half - 1)
            def _():
                fire(0, s + 2, sem0)

            drain(1, sem1)
            accum(1, s + 1)

            @pl.when(t < half - 1)
            def _():
                fire(1, s + 3, sem1)

            return carry

        lax.fori_loop(0, half, body, 0)
        pltpu.sync_copy(out_v, out_hbm.at[pl.ds(wid * rows_per_w, rows_per_w)])

    return k(flat_table, gidx2)


def kernel(categorical_vars_tensor, tables):
    F, V, D = tables.shape
    B = categorical_vars_tensor.shape[0]
    idx = categorical_vars_tensor.astype(jnp.int32)
    gidx = (idx + (jnp.arange(F, dtype=jnp.int32) * V)[None, :])
    gidx2 = gidx.reshape(B // CB, CB * F)
    flat_table = tables.reshape(F * V, D)
    return _lookup_mean(flat_table, gidx2, B=B, F=F, D=D)


# speedup vs baseline: 2.8688x; 2.5731x over previous
import functools

import jax
import jax.numpy as jnp
from jax import lax
from jax.experimental import pallas as pl
from jax.experimental.pallas import tpu as pltpu
from jax.experimental.pallas import tpu_sc as plsc

NUM_WORKERS = 32
ICH = 4096


@functools.partial(jax.jit, static_argnames=("B", "F", "V", "D"))
def _lookup_mean(table_t, idx_t, *, B, F, V, D):
    n_ich = B // ICH
    inv_f = jnp.float32(1.0 / F)

    mesh = plsc.VectorSubcoreMesh(core_axis_name="c", subcore_axis_name="s")

    @functools.partial(
        pl.kernel,
        mesh=mesh,
        compiler_params=pltpu.CompilerParams(needs_layout_passes=False),
        out_type=jax.ShapeDtypeStruct((D, B), jnp.float32),
        scratch_types=[
            pltpu.VMEM((V,), jnp.float32),
            pltpu.VMEM((B,), jnp.float32),
            pltpu.VMEM((2, ICH), jnp.int32),
            pltpu.SemaphoreType.DMA,
        ],
    )
    def k(table_hbm, idx_hbm, out_hbm, line_v, acc_v, idx_v, sem_i):
        d = lax.axis_index("s") * 2 + lax.axis_index("c")

        def zero_body(i, carry):
            acc_v[pl.ds(i * 16, 16)] = jnp.zeros((16,), jnp.float32)
            return carry

        lax.fori_loop(0, B // 16, zero_body, 0)

        def field_body(f, carry):
            pltpu.sync_copy(table_hbm.at[f, d], line_v)
            pltpu.async_copy(idx_hbm.at[f, pl.ds(0, ICH)], idx_v.at[0], sem_i)

            def chunk(c, buf, nxt_c, nxt_buf):
                pltpu.make_async_copy(
                    idx_hbm.at[f, pl.ds(0, ICH)], idx_v.at[buf], sem_i
                ).wait()

                @pl.when(nxt_c < n_ich)
                def _():
                    pltpu.async_copy(
                        idx_hbm.at[f, pl.ds(nxt_c * ICH, ICH)],
                        idx_v.at[nxt_buf],
                        sem_i,
                    )

                base = c * ICH

                def gat(i, carry2):
                    off = i * 64
                    for u in range(4):
                        ids = idx_v[buf, pl.ds(off + u * 16, 16)]
                        g = plsc.load_gather(line_v, [ids])
                        a = acc_v[pl.ds(base + off + u * 16, 16)]
                        acc_v[pl.ds(base + off + u * 16, 16)] = a + g
                    return carry2

                lax.fori_loop(0, ICH // 64, gat, 0)

            def chunk_pair(t, carry2):
                c = 2 * t
                chunk(c, 0, c + 1, 1)
                chunk(c + 1, 1, c + 2, 0)
                return carry2

            lax.fori_loop(0, n_ich // 2, chunk_pair, 0)
            return carry

        lax.fori_loop(0, F, field_body, 0)

        def scale_body(i, carry):
            acc_v[pl.ds(i * 16, 16)] = acc_v[pl.ds(i * 16, 16)] * inv_f
            return carry

        lax.fori_loop(0, B // 16, scale_body, 0)
        pltpu.sync_copy(acc_v, out_hbm.at[d])

    return k(table_t, idx_t)


def kernel(categorical_vars_tensor, tables):
    F, V, D = tables.shape
    B = categorical_vars_tensor.shape[0]
    idx_t = categorical_vars_tensor.astype(jnp.int32).T
    table_t = tables.transpose(0, 2, 1)
    out_t = _lookup_mean(table_t, idx_t, B=B, F=F, V=V, D=D)
    return out_t.T


# speedup vs baseline: 5.4518x; 1.9004x over previous
import functools

import jax
import jax.numpy as jnp
from jax import lax
from jax.experimental import pallas as pl
from jax.experimental.pallas import tpu as pltpu
from jax.experimental.pallas import tpu_sc as plsc

NUM_WORKERS = 32
ICH = 4096


@functools.partial(jax.jit, static_argnames=("B", "F", "V", "D"))
def _lookup_mean(table_t, idx_t, *, B, F, V, D):
    n_ich = B // ICH
    inv_f = jnp.float32(1.0 / F)

    mesh = plsc.VectorSubcoreMesh(core_axis_name="c", subcore_axis_name="s")

    @functools.partial(
        pl.kernel,
        mesh=mesh,
        compiler_params=pltpu.CompilerParams(needs_layout_passes=False),
        out_type=jax.ShapeDtypeStruct((D, B), jnp.float32),
        scratch_types=[
            pltpu.VMEM((V,), jnp.float32),
            pltpu.VMEM((B,), jnp.float32),
            pltpu.VMEM((2, ICH), jnp.int32),
            pltpu.SemaphoreType.DMA,
        ],
    )
    def k(table_hbm, idx_hbm, out_hbm, line_v, acc_v, idx_v, sem_i):
        d = lax.axis_index("s") * 2 + lax.axis_index("c")

        @plsc.parallel_loop(0, B // 16, unroll=8)
        def zero_body(i):
            acc_v[pl.ds(i * 16, 16)] = jnp.zeros((16,), jnp.float32)

        def field_body(f, carry):
            pltpu.sync_copy(table_hbm.at[f, d], line_v)
            pltpu.async_copy(idx_hbm.at[f, pl.ds(0, ICH)], idx_v.at[0], sem_i)

            def chunk(c, buf, nxt_c, nxt_buf):
                pltpu.make_async_copy(
                    idx_hbm.at[f, pl.ds(0, ICH)], idx_v.at[buf], sem_i
                ).wait()

                @pl.when(nxt_c < n_ich)
                def _():
                    pltpu.async_copy(
                        idx_hbm.at[f, pl.ds(nxt_c * ICH, ICH)],
                        idx_v.at[nxt_buf],
                        sem_i,
                    )

                base = c * ICH

                @plsc.parallel_loop(0, ICH // 16, unroll=8)
                def gat(i):
                    ids = idx_v[buf, pl.ds(i * 16, 16)]
                    g = plsc.load_gather(line_v, [ids])
                    a = acc_v[pl.ds(base + i * 16, 16)]
                    acc_v[pl.ds(base + i * 16, 16)] = a + g

            def chunk_pair(t, carry2):
                c = 2 * t
                chunk(c, 0, c + 1, 1)
                chunk(c + 1, 1, c + 2, 0)
                return carry2

            lax.fori_loop(0, n_ich // 2, chunk_pair, 0)
            return carry

        lax.fori_loop(0, F, field_body, 0)

        @plsc.parallel_loop(0, B // 16, unroll=8)
        def scale_body(i):
            acc_v[pl.ds(i * 16, 16)] = acc_v[pl.ds(i * 16, 16)] * inv_f
        pltpu.sync_copy(acc_v, out_hbm.at[d])

    return k(table_t, idx_t)


def kernel(categorical_vars_tensor, tables):
    F, V, D = tables.shape
    B = categorical_vars_tensor.shape[0]
    idx_t = categorical_vars_tensor.astype(jnp.int32).T
    table_t = tables.transpose(0, 2, 1)
    out_t = _lookup_mean(table_t, idx_t, B=B, F=F, V=V, D=D)
    return out_t.T
